# baseline (speedup 1.0000x reference)
import functools

import jax
import jax.numpy as jnp
from jax import lax
from jax.experimental import pallas as pl
from jax.experimental.pallas import tpu as pltpu
from jax.experimental.pallas import tpu_sc as plsc

NC = 2
NS = 16
LANES = 16
K = 40
ZB = 128
DEGW = 16
NB = 5


def _fill_loop(ref, rows, width, value):
  dt = ref.dtype
  lanes = 2 * LANES if dt == jnp.bfloat16 else LANES
  per_row = width // lanes

  def body(i, _):
    r = i // per_row
    g = i % per_row
    ref[r, pl.ds(g * lanes, lanes)] = jnp.full((lanes,), value, dt)
    return 0

  lax.fori_loop(0, rows * per_row, body, 0)


def _build_deg_kernel(n_pad, n_chunks):
  rows_per_sub = n_pad // NS
  chunks_per_worker = n_chunks // (NC * NS)
  mesh = plsc.VectorSubcoreMesh(core_axis_name="c", subcore_axis_name="s")
  out_t = (jax.ShapeDtypeStruct((n_pad, DEGW), jnp.float32),
           jax.ShapeDtypeStruct((n_pad, DEGW), jnp.float32))
  scratch = [
      pltpu.VMEM((chunks_per_worker, K), jnp.int32),
      pltpu.VMEM((K, DEGW), jnp.float32),
      pltpu.VMEM((ZB, DEGW), jnp.float32),
      pltpu.VMEM_SHARED((n_pad, DEGW), jnp.float32),
  ]

  @functools.partial(
      pl.kernel, out_type=out_t, mesh=mesh, scratch_types=scratch,
      compiler_params=pltpu.CompilerParams(use_tc_tiling_on_sc=False))
  def deg_kernel(dst3, deg0, deg1, dstv, ones, zbuf, acc):
    c = lax.axis_index("c")
    s = lax.axis_index("s")
    wid = s * NC + c
    _fill_loop(ones, K, DEGW, 1.0)
    _fill_loop(zbuf, ZB, DEGW, 0.0)
    row_base = s * rows_per_sub

    def zcopy(i, _):
      pltpu.sync_copy(zbuf, acc.at[pl.ds(row_base + i * ZB, ZB)])
      return 0

    lax.fori_loop(0, rows_per_sub // ZB, zcopy, 0)
    pltpu.sync_copy(dst3.at[wid], dstv)
    plsc.subcore_barrier()

    def body(j, _):
      pltpu.sync_copy(ones, acc.at[dstv.at[j]], add=True)
      return 0

    lax.fori_loop(0, chunks_per_worker, body, 0)
    plsc.subcore_barrier()

    @pl.when(c == 0)
    def _():
      pltpu.sync_copy(acc.at[pl.ds(row_base, rows_per_sub)],
                      deg0.at[pl.ds(row_base, rows_per_sub)])

    @pl.when(c == 1)
    def _():
      pltpu.sync_copy(acc.at[pl.ds(row_base, rows_per_sub)],
                      deg1.at[pl.ds(row_base, rows_per_sub)])

  return deg_kernel


def _build_segsum_kernel(n, n_pad, dhalf, n_chunks, dt):
  rows_per_sub = n_pad // NS
  chunks_per_sub = n_chunks // NS
  assert chunks_per_sub % NB == 0
  mesh = plsc.VectorSubcoreMesh(core_axis_name="c", subcore_axis_name="s")
  out_t = (jax.ShapeDtypeStruct((n_pad, dhalf), dt),
           jax.ShapeDtypeStruct((n_pad, dhalf), dt))
  scratch = [
      pltpu.VMEM((chunks_per_sub, K), jnp.int32),
      pltpu.VMEM((chunks_per_sub, K), jnp.int32),
      [pltpu.VMEM((K, dhalf), dt)] * NB,
      pltpu.VMEM((ZB, dhalf), dt),
      pltpu.VMEM_SHARED((n_pad, dhalf), dt),
      [pltpu.SemaphoreType.DMA] * NB,
  ]

  @functools.partial(
      pl.kernel, out_type=out_t, mesh=mesh, scratch_types=scratch,
      compiler_params=pltpu.CompilerParams(use_tc_tiling_on_sc=False))
  def segsum_kernel(pL, pR, src3, dst3, segL, segR,
                    srcv, dstv, bufs, zbuf, acc, sems):
    c = lax.axis_index("c")
    s = lax.axis_index("s")
    _fill_loop(zbuf, ZB, dhalf, 0.0)
    row_base = s * rows_per_sub

    def zcopy(i, _):
      pltpu.sync_copy(zbuf, acc.at[pl.ds(row_base + i * ZB, ZB)])
      return 0

    lax.fori_loop(0, rows_per_sub // ZB, zcopy, 0)
    pltpu.sync_copy(src3.at[s], srcv)
    pltpu.sync_copy(dst3.at[s], dstv)
    plsc.subcore_barrier()

    def core_body(p_ref):
      for b in range(NB):
        pltpu.async_copy(p_ref.at[srcv.at[b]], bufs[b], sems[b])

      def body(it, _):
        g = it * NB
        for b in range(NB):
          j = g + b
          pltpu.make_async_copy(p_ref.at[srcv.at[j]], bufs[b],
                                sems[b]).wait()
          pltpu.sync_copy(bufs[b], acc.at[dstv.at[j]], add=True)

          @pl.when(j + NB < chunks_per_sub)
          def _(b=b, j=j):
            pltpu.async_copy(p_ref.at[srcv.at[j + NB]], bufs[b], sems[b])

        return 0

      lax.fori_loop(0, chunks_per_sub // NB, body, 0)

    @pl.when(c == 0)
    def _():
      core_body(pL)

    @pl.when(c == 1)
    def _():
      core_body(pR)

    plsc.subcore_barrier()

    @pl.when(c == 0)
    def _():
      pltpu.sync_copy(acc.at[pl.ds(row_base, rows_per_sub)],
                      segL.at[pl.ds(row_base, rows_per_sub)])

    @pl.when(c == 1)
    def _():
      pltpu.sync_copy(acc.at[pl.ds(row_base, rows_per_sub)],
                      segR.at[pl.ds(row_base, rows_per_sub)])

  return segsum_kernel


def _prep_body(x_ref, d0_ref, d1_ref, w_ref, pl_ref, pr_ref, dis_ref):
  deg = jnp.sum(d0_ref[...] + d1_ref[...], axis=1, keepdims=True) / DEGW + 1.0
  dis = lax.rsqrt(deg)
  p = jnp.dot(x_ref[...], w_ref[...], preferred_element_type=jnp.float32) * dis
  half = p.shape[1] // 2
  pl_ref[...] = p[:, :half].astype(pl_ref.dtype)
  pr_ref[...] = p[:, half:].astype(pr_ref.dtype)
  dis_ref[...] = dis


def _layer_body(sl_ref, sr_ref, pl_in_ref, pr_in_ref, dis_ref, b_ref, w_ref,
                h_ref, pl_ref, pr_ref):
  dis = dis_ref[...]
  f32 = jnp.float32
  agg = jnp.concatenate(
      [sl_ref[...].astype(f32) + pl_in_ref[...].astype(f32),
       sr_ref[...].astype(f32) + pr_in_ref[...].astype(f32)], axis=1)
  h = jnp.maximum(dis * agg + b_ref[...], 0.0)
  h_ref[...] = h
  p = jnp.dot(h, w_ref[...], preferred_element_type=jnp.float32) * dis
  half = p.shape[1] // 2
  pl_ref[...] = p[:, :half].astype(pl_ref.dtype)
  pr_ref[...] = p[:, half:].astype(pr_ref.dtype)


def _final_body(sl_ref, sr_ref, pl_in_ref, pr_in_ref, dis_ref, b_ref,
                x_ref, h1_ref, h2_ref, wr_ref, br_ref, out_ref):
  dis = dis_ref[...]
  f32 = jnp.float32
  agg = jnp.concatenate(
      [sl_ref[...].astype(f32) + pl_in_ref[...].astype(f32),
       sr_ref[...].astype(f32) + pr_in_ref[...].astype(f32)], axis=1)
  h3 = jnp.maximum(dis * agg + b_ref[...], 0.0)
  d = x_ref.shape[1]
  wr = wr_ref[...]
  acc = jnp.dot(x_ref[...], wr[:d], preferred_element_type=jnp.float32)
  acc += jnp.dot(h1_ref[...], wr[d:2 * d], preferred_element_type=jnp.float32)
  acc += jnp.dot(h2_ref[...], wr[2 * d:3 * d],
                 preferred_element_type=jnp.float32)
  acc += jnp.dot(h3, wr[3 * d:], preferred_element_type=jnp.float32)
  out_ref[...] = acc + br_ref[...]


def _row_spec(r, cols):
  return pl.BlockSpec((r, cols), lambda i: (i, 0))


def _whole_spec(shape):
  return pl.BlockSpec(shape, lambda i: tuple(0 for _ in shape))


def kernel(x, edge_index, W0, b0, W1, b1, W2, b2, Wr, br):
  n, d = x.shape
  e = edge_index.shape[1]
  h_dim = W0.shape[1]
  c_out = Wr.shape[1]
  dhalf = h_dim // 2

  rows_per_sub = -(-n // (NS * ZB)) * ZB
  n_pad = NS * rows_per_sub

  epw = K * NC * NS * NB
  e_pad = -(-e // epw) * epw
  if e_pad != e:
    npad_rows = n_pad - n
    dummy_dst = n + jax.lax.iota(jnp.int32, e_pad - e) % npad_rows
    dummy = jnp.stack(
        [jnp.zeros((e_pad - e,), jnp.int32), dummy_dst], axis=0)
    edge_index = jnp.concatenate([edge_index, dummy], axis=1)
  n_chunks = e_pad // K

  src_ss = edge_index[0].reshape(NS, n_chunks // NS, K)
  dst_ss = edge_index[1].reshape(NS, n_chunks // NS, K)
  dst_dg = edge_index[1].reshape(NC * NS, n_chunks // (NC * NS), K)

  deg_kernel = _build_deg_kernel(n_pad, n_chunks)
  seg_dt = jnp.bfloat16
  segsum_kernel = _build_segsum_kernel(n, n_pad, dhalf, n_chunks, seg_dt)

  R = 2000
  grid = (n // R,)

  prep = pl.pallas_call(
      _prep_body,
      grid=grid,
      in_specs=[_row_spec(R, d), _row_spec(R, DEGW), _row_spec(R, DEGW),
                _whole_spec((d, h_dim))],
      out_specs=[_row_spec(R, dhalf), _row_spec(R, dhalf), _row_spec(R, 1)],
      out_shape=[jax.ShapeDtypeStruct((n, dhalf), seg_dt),
                 jax.ShapeDtypeStruct((n, dhalf), seg_dt),
                 jax.ShapeDtypeStruct((n, 1), jnp.float32)],
  )

  layer = pl.pallas_call(
      _layer_body,
      grid=grid,
      in_specs=[_row_spec(R, dhalf), _row_spec(R, dhalf),
                _row_spec(R, dhalf), _row_spec(R, dhalf),
                _row_spec(R, 1), _whole_spec((1, h_dim)),
                _whole_spec((h_dim, h_dim))],
      out_specs=[_row_spec(R, h_dim), _row_spec(R, dhalf),
                 _row_spec(R, dhalf)],
      out_shape=[jax.ShapeDtypeStruct((n, h_dim), jnp.float32),
                 jax.ShapeDtypeStruct((n, dhalf), seg_dt),
                 jax.ShapeDtypeStruct((n, dhalf), seg_dt)],
  )

  final = pl.pallas_call(
      _final_body,
      grid=grid,
      in_specs=[_row_spec(R, dhalf), _row_spec(R, dhalf),
                _row_spec(R, dhalf), _row_spec(R, dhalf),
                _row_spec(R, 1), _whole_spec((1, h_dim)),
                _row_spec(R, d), _row_spec(R, h_dim), _row_spec(R, h_dim),
                _whole_spec((d + 3 * h_dim, c_out)),
                _whole_spec((1, c_out))],
      out_specs=[_row_spec(R, c_out)],
      out_shape=[jax.ShapeDtypeStruct((n, c_out), jnp.float32)],
  )

  deg0, deg1 = deg_kernel(dst_dg)
  pL, pR, dis = prep(x, deg0, deg1, W0)
  b0r = b0.reshape(1, h_dim)
  b1r = b1.reshape(1, h_dim)
  b2r = b2.reshape(1, h_dim)
  brr = br.reshape(1, c_out)

  segL, segR = segsum_kernel(pL, pR, src_ss, dst_ss)
  h1, pL, pR = layer(segL, segR, pL, pR, dis, b0r, W1)
  segL, segR = segsum_kernel(pL, pR, src_ss, dst_ss)
  h2, pL, pR = layer(segL, segR, pL, pR, dis, b1r, W2)
  segL, segR = segsum_kernel(pL, pR, src_ss, dst_ss)
  (out,) = final(segL, segR, pL, pR, dis, b2r, x, h1, h2, Wr, brr)
  return out


# speedup vs baseline: 1.2723x; 1.2723x over previous
import functools

import jax
import jax.numpy as jnp
from jax import lax
from jax.experimental import pallas as pl
from jax.experimental.pallas import tpu as pltpu
from jax.experimental.pallas import tpu_sc as plsc

NC = 2
NS = 16
LANES = 16
K = 80
ZB = 128
DEGW = 16
NB = 5


def _fill_loop(ref, rows, width, value):
  dt = ref.dtype
  lanes = 2 * LANES if dt == jnp.bfloat16 else LANES
  per_row = width // lanes

  def body(i, _):
    r = i // per_row
    g = i % per_row
    ref[r, pl.ds(g * lanes, lanes)] = jnp.full((lanes,), value, dt)
    return 0

  lax.fori_loop(0, rows * per_row, body, 0)


def _build_deg_kernel(n_pad, n_chunks):
  rows_per_sub = n_pad // NS
  chunks_per_worker = n_chunks // (NC * NS)
  mesh = plsc.VectorSubcoreMesh(core_axis_name="c", subcore_axis_name="s")
  out_t = (jax.ShapeDtypeStruct((n_pad, DEGW), jnp.float32),
           jax.ShapeDtypeStruct((n_pad, DEGW), jnp.float32))
  scratch = [
      pltpu.VMEM((chunks_per_worker, K), jnp.int32),
      pltpu.VMEM((K, DEGW), jnp.float32),
      pltpu.VMEM((ZB, DEGW), jnp.float32),
      pltpu.VMEM_SHARED((n_pad, DEGW), jnp.float32),
  ]

  @functools.partial(
      pl.kernel, out_type=out_t, mesh=mesh, scratch_types=scratch,
      compiler_params=pltpu.CompilerParams(use_tc_tiling_on_sc=False))
  def deg_kernel(dst3, deg0, deg1, dstv, ones, zbuf, acc):
    c = lax.axis_index("c")
    s = lax.axis_index("s")
    wid = s * NC + c
    _fill_loop(ones, K, DEGW, 1.0)
    _fill_loop(zbuf, ZB, DEGW, 0.0)
    row_base = s * rows_per_sub

    def zcopy(i, _):
      pltpu.sync_copy(zbuf, acc.at[pl.ds(row_base + i * ZB, ZB)])
      return 0

    lax.fori_loop(0, rows_per_sub // ZB, zcopy, 0)
    pltpu.sync_copy(dst3.at[wid], dstv)
    plsc.subcore_barrier()

    def body(j, _):
      pltpu.sync_copy(ones, acc.at[dstv.at[j]], add=True)
      return 0

    lax.fori_loop(0, chunks_per_worker, body, 0)
    plsc.subcore_barrier()

    @pl.when(c == 0)
    def _():
      pltpu.sync_copy(acc.at[pl.ds(row_base, rows_per_sub)],
                      deg0.at[pl.ds(row_base, rows_per_sub)])

    @pl.when(c == 1)
    def _():
      pltpu.sync_copy(acc.at[pl.ds(row_base, rows_per_sub)],
                      deg1.at[pl.ds(row_base, rows_per_sub)])

  return deg_kernel


def _build_segsum_kernel(n, n_pad, dhalf, n_chunks, dt):
  rows_per_sub = n_pad // NS
  chunks_per_sub = n_chunks // NS
  assert chunks_per_sub % NB == 0
  mesh = plsc.VectorSubcoreMesh(core_axis_name="c", subcore_axis_name="s")
  out_t = (jax.ShapeDtypeStruct((n_pad, dhalf), dt),
           jax.ShapeDtypeStruct((n_pad, dhalf), dt))
  scratch = [
      pltpu.VMEM((chunks_per_sub, K), jnp.int32),
      pltpu.VMEM((chunks_per_sub, K), jnp.int32),
      [pltpu.VMEM((K, dhalf), dt)] * NB,
      pltpu.VMEM((ZB, dhalf), dt),
      pltpu.VMEM_SHARED((n_pad, dhalf), dt),
      [pltpu.SemaphoreType.DMA] * NB,
  ]

  @functools.partial(
      pl.kernel, out_type=out_t, mesh=mesh, scratch_types=scratch,
      compiler_params=pltpu.CompilerParams(use_tc_tiling_on_sc=False))
  def segsum_kernel(pL, pR, src3, dst3, segL, segR,
                    srcv, dstv, bufs, zbuf, acc, sems):
    c = lax.axis_index("c")
    s = lax.axis_index("s")
    _fill_loop(zbuf, ZB, dhalf, 0.0)
    row_base = s * rows_per_sub

    def zcopy(i, _):
      pltpu.sync_copy(zbuf, acc.at[pl.ds(row_base + i * ZB, ZB)])
      return 0

    lax.fori_loop(0, rows_per_sub // ZB, zcopy, 0)
    pltpu.sync_copy(src3.at[s], srcv)
    pltpu.sync_copy(dst3.at[s], dstv)
    plsc.subcore_barrier()

    def core_body(p_ref):
      for b in range(NB):
        pltpu.async_copy(p_ref.at[srcv.at[b]], bufs[b], sems[b])

      def body(it, _):
        g = it * NB
        for b in range(NB):
          j = g + b
          pltpu.make_async_copy(p_ref.at[srcv.at[j]], bufs[b],
                                sems[b]).wait()
          pltpu.sync_copy(bufs[b], acc.at[dstv.at[j]], add=True)

          @pl.when(j + NB < chunks_per_sub)
          def _(b=b, j=j):
            pltpu.async_copy(p_ref.at[srcv.at[j + NB]], bufs[b], sems[b])

        return 0

      lax.fori_loop(0, chunks_per_sub // NB, body, 0)

    @pl.when(c == 0)
    def _():
      core_body(pL)

    @pl.when(c == 1)
    def _():
      core_body(pR)

    plsc.subcore_barrier()

    @pl.when(c == 0)
    def _():
      pltpu.sync_copy(acc.at[pl.ds(row_base, rows_per_sub)],
                      segL.at[pl.ds(row_base, rows_per_sub)])

    @pl.when(c == 1)
    def _():
      pltpu.sync_copy(acc.at[pl.ds(row_base, rows_per_sub)],
                      segR.at[pl.ds(row_base, rows_per_sub)])

  return segsum_kernel


def _prep_body(x_ref, d0_ref, d1_ref, w_ref, pl_ref, pr_ref, dis_ref):
  deg = jnp.sum(d0_ref[...] + d1_ref[...], axis=1, keepdims=True) / DEGW + 1.0
  dis = lax.rsqrt(deg)
  p = jnp.dot(x_ref[...], w_ref[...], preferred_element_type=jnp.float32) * dis
  half = p.shape[1] // 2
  pl_ref[...] = p[:, :half].astype(pl_ref.dtype)
  pr_ref[...] = p[:, half:].astype(pr_ref.dtype)
  dis_ref[...] = dis


def _layer_body(sl_ref, sr_ref, pl_in_ref, pr_in_ref, dis_ref, b_ref, w_ref,
                h_ref, pl_ref, pr_ref):
  dis = dis_ref[...]
  f32 = jnp.float32
  agg = jnp.concatenate(
      [sl_ref[...].astype(f32) + pl_in_ref[...].astype(f32),
       sr_ref[...].astype(f32) + pr_in_ref[...].astype(f32)], axis=1)
  h = jnp.maximum(dis * agg + b_ref[...], 0.0)
  h_ref[...] = h
  p = jnp.dot(h, w_ref[...], preferred_element_type=jnp.float32) * dis
  half = p.shape[1] // 2
  pl_ref[...] = p[:, :half].astype(pl_ref.dtype)
  pr_ref[...] = p[:, half:].astype(pr_ref.dtype)


def _final_body(sl_ref, sr_ref, pl_in_ref, pr_in_ref, dis_ref, b_ref,
                x_ref, h1_ref, h2_ref, wr_ref, br_ref, out_ref):
  dis = dis_ref[...]
  f32 = jnp.float32
  agg = jnp.concatenate(
      [sl_ref[...].astype(f32) + pl_in_ref[...].astype(f32),
       sr_ref[...].astype(f32) + pr_in_ref[...].astype(f32)], axis=1)
  h3 = jnp.maximum(dis * agg + b_ref[...], 0.0)
  d = x_ref.shape[1]
  wr = wr_ref[...]
  acc = jnp.dot(x_ref[...], wr[:d], preferred_element_type=jnp.float32)
  acc += jnp.dot(h1_ref[...], wr[d:2 * d], preferred_element_type=jnp.float32)
  acc += jnp.dot(h2_ref[...], wr[2 * d:3 * d],
                 preferred_element_type=jnp.float32)
  acc += jnp.dot(h3, wr[3 * d:], preferred_element_type=jnp.float32)
  out_ref[...] = acc + br_ref[...]


def _row_spec(r, cols):
  return pl.BlockSpec((r, cols), lambda i: (i, 0))


def _whole_spec(shape):
  return pl.BlockSpec(shape, lambda i: tuple(0 for _ in shape))


def kernel(x, edge_index, W0, b0, W1, b1, W2, b2, Wr, br):
  n, d = x.shape
  e = edge_index.shape[1]
  h_dim = W0.shape[1]
  c_out = Wr.shape[1]
  dhalf = h_dim // 2

  rows_per_sub = -(-n // (NS * ZB)) * ZB
  n_pad = NS * rows_per_sub

  epw = K * NC * NS * NB
  e_pad = -(-e // epw) * epw
  if e_pad != e:
    npad_rows = n_pad - n
    dummy_dst = n + jax.lax.iota(jnp.int32, e_pad - e) % npad_rows
    dummy = jnp.stack(
        [jnp.zeros((e_pad - e,), jnp.int32), dummy_dst], axis=0)
    edge_index = jnp.concatenate([edge_index, dummy], axis=1)
  n_chunks = e_pad // K

  src_ss = edge_index[0].reshape(NS, n_chunks // NS, K)
  dst_ss = edge_index[1].reshape(NS, n_chunks // NS, K)
  dst_dg = edge_index[1].reshape(NC * NS, n_chunks // (NC * NS), K)

  deg_kernel = _build_deg_kernel(n_pad, n_chunks)
  seg_dt = jnp.bfloat16
  segsum_kernel = _build_segsum_kernel(n, n_pad, dhalf, n_chunks, seg_dt)

  R = 2000
  grid = (n // R,)

  prep = pl.pallas_call(
      _prep_body,
      grid=grid,
      in_specs=[_row_spec(R, d), _row_spec(R, DEGW), _row_spec(R, DEGW),
                _whole_spec((d, h_dim))],
      out_specs=[_row_spec(R, dhalf), _row_spec(R, dhalf), _row_spec(R, 1)],
      out_shape=[jax.ShapeDtypeStruct((n, dhalf), seg_dt),
                 jax.ShapeDtypeStruct((n, dhalf), seg_dt),
                 jax.ShapeDtypeStruct((n, 1), jnp.float32)],
  )

  layer = pl.pallas_call(
      _layer_body,
      grid=grid,
      in_specs=[_row_spec(R, dhalf), _row_spec(R, dhalf),
                _row_spec(R, dhalf), _row_spec(R, dhalf),
                _row_spec(R, 1), _whole_spec((1, h_dim)),
                _whole_spec((h_dim, h_dim))],
      out_specs=[_row_spec(R, h_dim), _row_spec(R, dhalf),
                 _row_spec(R, dhalf)],
      out_shape=[jax.ShapeDtypeStruct((n, h_dim), jnp.float32),
                 jax.ShapeDtypeStruct((n, dhalf), seg_dt),
                 jax.ShapeDtypeStruct((n, dhalf), seg_dt)],
  )

  final = pl.pallas_call(
      _final_body,
      grid=grid,
      in_specs=[_row_spec(R, dhalf), _row_spec(R, dhalf),
                _row_spec(R, dhalf), _row_spec(R, dhalf),
                _row_spec(R, 1), _whole_spec((1, h_dim)),
                _row_spec(R, d), _row_spec(R, h_dim), _row_spec(R, h_dim),
                _whole_spec((d + 3 * h_dim, c_out)),
                _whole_spec((1, c_out))],
      out_specs=[_row_spec(R, c_out)],
      out_shape=[jax.ShapeDtypeStruct((n, c_out), jnp.float32)],
  )

  deg0, deg1 = deg_kernel(dst_dg)
  pL, pR, dis = prep(x, deg0, deg1, W0)
  b0r = b0.reshape(1, h_dim)
  b1r = b1.reshape(1, h_dim)
  b2r = b2.reshape(1, h_dim)
  brr = br.reshape(1, c_out)

  segL, segR = segsum_kernel(pL, pR, src_ss, dst_ss)
  h1, pL, pR = layer(segL, segR, pL, pR, dis, b0r, W1)
  segL, segR = segsum_kernel(pL, pR, src_ss, dst_ss)
  h2, pL, pR = layer(segL, segR, pL, pR, dis, b1r, W2)
  segL, segR = segsum_kernel(pL, pR, src_ss, dst_ss)
  (out,) = final(segL, segR, pL, pR, dis, b2r, x, h1, h2, Wr, brr)
  return out
